# baseline (speedup 1.0000x reference)
import functools

import jax
import jax.numpy as jnp
from jax import lax
from jax.experimental import pallas as pl
from jax.experimental.pallas import tpu as pltpu
from jax.experimental.pallas import tpu_sc as plsc

N = 10000
E = 320000
NE = 10000
NNZ = 320000
D = 128

NC = 2
NS = 16
NW = NC * NS
C = 64
NP = 10240
RPT = NP // NS
SINK = NP - 1
EPT = E // NW
EPTP = 10240
_PH = 5
CPH = EPTP // _PH // C
NB = 4

@functools.lru_cache(maxsize=None)
def _mesh():
    return plsc.VectorSubcoreMesh(core_axis_name="c", subcore_axis_name="s",
                                  num_cores=NC, num_subcores=NS)


@functools.lru_cache(maxsize=None)
def _make_segsum(T, E2):
    @functools.partial(
        pl.kernel,
        out_type=jax.ShapeDtypeStruct((NC, NP, D), jnp.float32),
        mesh=_mesh(),
        scratch_types=[
            pltpu.VMEM_SHARED((NP, D), jnp.float32),
            pltpu.VMEM((CPH, C), jnp.int32),
            pltpu.VMEM((CPH, C), jnp.int32),
        ] + [pltpu.VMEM((C, D), jnp.float32)] * NB
          + [pltpu.SemaphoreType.DMA] * (2 * NB),
    )
    def segsum(table, src4, dst4, zeros, out,
               acc, sidx, didx, r0, r1, r2, r3,
               g0, g1, g2, g3, s0, s1, s2, s3):
        c = lax.axis_index("c")
        s = lax.axis_index("s")
        w = c * NS + s
        pltpu.sync_copy(zeros, acc.at[pl.ds(s * RPT, RPT)])
        plsc.subcore_barrier()

        rows = (r0, r1, r2, r3)
        gsem = (g0, g1, g2, g3)
        ssem = (s0, s1, s2, s3)
        for phase in range(_PH):
            pltpu.sync_copy(src4.at[w, phase], sidx)
            pltpu.sync_copy(dst4.at[w, phase], didx)
            for b in range(NB):
                pltpu.async_copy(table.at[sidx.at[b]], rows[b], gsem[b])

            def body(i, carry):
                for b in range(NB):
                    j = NB * i + b
                    g = j + 2
                    bg = (b + 2) % NB

                    @pl.when(jnp.logical_and(g >= NB, g < CPH))
                    def _():
                        pltpu.make_async_copy(
                            rows[bg], acc.at[didx.at[g - NB]],
                            ssem[bg]).wait()
                        pltpu.async_copy(table.at[sidx.at[g]],
                                         rows[bg], gsem[bg])

                    pltpu.make_async_copy(table.at[sidx.at[j]],
                                          rows[b], gsem[b]).wait()
                    pltpu.async_copy(rows[b], acc.at[didx.at[j]],
                                     ssem[b], add=True)
                return carry

            lax.fori_loop(0, CPH // NB, body, 0)
            for b in range(NB):
                pltpu.make_async_copy(rows[b], acc.at[didx.at[CPH - NB + b]],
                                      ssem[b]).wait()
        plsc.subcore_barrier()
        pltpu.sync_copy(acc.at[pl.ds(s * RPT, RPT)],
                        out.at[c, pl.ds(s * RPT, RPT)])

    return segsum


_CW = 128
_CCH = (E // NW) // C


@functools.lru_cache(maxsize=None)
def _make_counts():
    @functools.partial(
        pl.kernel,
        out_type=(
            jax.ShapeDtypeStruct((NC, NP, _CW), jnp.float32),
            jax.ShapeDtypeStruct((NC, NP, _CW), jnp.float32),
            jax.ShapeDtypeStruct((NC, NP, _CW), jnp.float32),
        ),
        mesh=_mesh(),
        scratch_types=[
            pltpu.VMEM_SHARED((NP, _CW), jnp.float32),
            pltpu.VMEM((CPH, C), jnp.int32),
            pltpu.VMEM((C, _CW), jnp.float32),
            pltpu.SemaphoreType.DMA,
        ],
    )
    def counts(dst4, v4, e4, zeros, ones, outd, outv, oute,
               acc, idx, ones_v, sem):
        c = lax.axis_index("c")
        s = lax.axis_index("s")
        w = c * NS + s
        sl = pl.ds(s * RPT, RPT)
        pltpu.sync_copy(ones, ones_v)
        for idx4, out in ((dst4, outd), (v4, outv), (e4, oute)):
            pltpu.sync_copy(zeros, acc.at[sl])
            plsc.subcore_barrier()
            for phase in range(_PH):
                pltpu.sync_copy(idx4.at[w, phase], idx)

                def body(j, carry):
                    @pl.when(j >= 4)
                    def _():
                        pltpu.make_async_copy(ones_v, acc.at[idx.at[j - 4]],
                                              sem).wait()

                    pltpu.async_copy(ones_v, acc.at[idx.at[j]], sem,
                                     add=True)
                    return carry

                lax.fori_loop(0, CPH, body, 0)
                for b in range(4):
                    pltpu.make_async_copy(ones_v, acc.at[idx.at[CPH - 4 + b]],
                                          sem).wait()
            plsc.subcore_barrier()
            pltpu.sync_copy(acc.at[sl], out.at[c, sl])
            plsc.subcore_barrier()

    return counts


_B = 1000
_GRID = N // _B


def _row_spec(nrow=_B, ncol=D):
    return pl.BlockSpec((nrow, ncol), lambda i: (i, 0))


def _part_spec(ncol=D):
    return pl.BlockSpec((NC, _B, ncol), lambda i: (0, i, 0))


def _full_spec(shape):
    nd = len(shape)
    return pl.BlockSpec(shape, lambda i: (0,) * nd)


def _tc1_body(x, wg1, wh1, bh1, degp, vcp, ecp,
              hg1p, hh1, dinv_b, vinv_b, einv_b):
    deg = degp[0, :, :1] + degp[1, :, :1] + 1.0
    dinv = lax.rsqrt(deg)
    dinv_b[...] = jnp.broadcast_to(dinv, (_B, D))
    vinv_b[...] = jnp.broadcast_to(
        1.0 / jnp.maximum(vcp[0, :, :1] + vcp[1, :, :1], 1.0), (_B, D))
    einv_b[...] = jnp.broadcast_to(
        1.0 / jnp.maximum(ecp[0, :, :1] + ecp[1, :, :1], 1.0), (_B, D))
    hg1p[...] = jnp.dot(x[...], wg1[...],
                        preferred_element_type=jnp.float32) * dinv_b[...]
    hh1[...] = jnp.dot(x[...], wh1[...],
                       preferred_element_type=jnp.float32) + bh1[...]


def _tc2_body(sg1p, hg1p, bg1, dinv_b, se1p, einv_b, wg2, hg2p, ef1):
    x1 = jnp.maximum(
        dinv_b[...] * (sg1p[0] + sg1p[1] + hg1p[...]) + bg1[...], 0.0)
    hg2p[...] = jnp.dot(x1, wg2[...],
                        preferred_element_type=jnp.float32) * dinv_b[...]
    ef1[...] = (se1p[0] + se1p[1]) * einv_b[...]


def _tc3_body(sg2p, hg2p, bg2, dinv_b, sv1p, vinv_b, wh2, bh2, x2, hh2):
    x2[...] = dinv_b[...] * (sg2p[0] + sg2p[1] + hg2p[...]) + bg2[...]
    x3 = jnp.maximum((sv1p[0] + sv1p[1]) * vinv_b[...], 0.0)
    hh2[...] = jnp.dot(x3, wh2[...],
                       preferred_element_type=jnp.float32) + bh2[...]


def _tc4_body(se2p, einv_b, ef2):
    ef2[...] = (se2p[0] + se2p[1]) * einv_b[...]


def _tc5_body(sv2p, vinv_b, x2, out):
    out[...] = 0.5 * x2[...] + 0.5 * (sv2p[0] + sv2p[1]) * vinv_b[...]


def _row_out(k=1):
    o = [jax.ShapeDtypeStruct((N, D), jnp.float32) for _ in range(k)]
    return o[0] if k == 1 else tuple(o)


_tc1 = pl.pallas_call(
    _tc1_body,
    grid=(_GRID,),
    in_specs=[_row_spec(), _full_spec((D, D)), _full_spec((D, D)),
              _full_spec((D,)), _part_spec(_CW), _part_spec(_CW),
              _part_spec(_CW)],
    out_specs=[_row_spec()] * 5,
    out_shape=_row_out(5),
)

_tc2 = pl.pallas_call(
    _tc2_body,
    grid=(_GRID,),
    in_specs=[_part_spec(), _row_spec(), _full_spec((D,)), _row_spec(),
              _part_spec(), _row_spec(), _full_spec((D, D))],
    out_specs=[_row_spec()] * 2,
    out_shape=_row_out(2),
)

_tc3 = pl.pallas_call(
    _tc3_body,
    grid=(_GRID,),
    in_specs=[_part_spec(), _row_spec(), _full_spec((D,)), _row_spec(),
              _part_spec(), _row_spec(), _full_spec((D, D)),
              _full_spec((D,))],
    out_specs=[_row_spec()] * 2,
    out_shape=_row_out(2),
)

_tc4 = pl.pallas_call(
    _tc4_body,
    grid=(_GRID,),
    in_specs=[_part_spec(), _row_spec()],
    out_specs=_row_spec(),
    out_shape=_row_out(),
)

_tc5 = pl.pallas_call(
    _tc5_body,
    grid=(_GRID,),
    in_specs=[_part_spec(), _row_spec(), _row_spec()],
    out_specs=_row_spec(),
    out_shape=_row_out(),
)


def kernel(x, edge_index, hyperedge_index,
           W_g1, b_g1, W_g2, b_g2, W_h1, b_h1, W_h2, b_h2):
    def chunked(a, fill):
        a = a.reshape(NW, EPT)
        pad = jnp.full((NW, EPTP - EPT), fill, a.dtype)
        return jnp.concatenate([a, pad], axis=1).reshape(NW, _PH, CPH, C)

    src2 = chunked(edge_index[0], 0)
    dst2 = chunked(edge_index[1], SINK)
    v2g = chunked(hyperedge_index[0], 0)
    v2s = chunked(hyperedge_index[0], SINK)
    e2g = chunked(hyperedge_index[1], 0)
    e2s = chunked(hyperedge_index[1], SINK)

    zeros_d = jnp.zeros((RPT, D), jnp.float32)
    zeros_c = jnp.zeros((RPT, _CW), jnp.float32)
    ones_c = jnp.ones((C, _CW), jnp.float32)

    segsum_nodes = _make_segsum(N, E)
    segsum_v2e = _make_segsum(N, NNZ)
    segsum_e2v = _make_segsum(NE, NNZ)

    degp, vcp, ecp = _make_counts()(dst2, v2s, e2s, zeros_c, ones_c)
    hg1p, hh1, dinv_b, vinv_b, einv_b = _tc1(
        x, W_g1, W_h1, b_h1, degp, vcp, ecp)

    sg1p = segsum_nodes(hg1p, src2, dst2, zeros_d)
    se1p = segsum_v2e(hh1, v2g, e2s, zeros_d)
    hg2p, ef1 = _tc2(sg1p, hg1p, b_g1, dinv_b, se1p, einv_b, W_g2)

    sg2p = segsum_nodes(hg2p, src2, dst2, zeros_d)
    sv1p = segsum_e2v(ef1, e2g, v2s, zeros_d)
    x2, hh2 = _tc3(sg2p, hg2p, b_g2, dinv_b, sv1p, vinv_b, W_h2, b_h2)

    se2p = segsum_v2e(hh2, v2g, e2s, zeros_d)
    ef2 = _tc4(se2p, einv_b)

    sv2p = segsum_e2v(ef2, e2g, v2s, zeros_d)
    return _tc5(sv2p, vinv_b, x2)


# speedup vs baseline: 2.7364x; 2.7364x over previous
import functools

import jax
import jax.numpy as jnp
from jax import lax
from jax.experimental import pallas as pl
from jax.experimental.pallas import tpu as pltpu
from jax.experimental.pallas import tpu_sc as plsc

N = 10000
E = 320000
NE = 10000
NNZ = 320000
D = 128

NC = 2
NS = 16
NW = NC * NS
C = 125
NP = 10240
RPT = NP // NS
SINK = NP - 1
EPT = E // NW
EPTP = 10000
_PH = 2
CPH = EPTP // _PH // C

@functools.lru_cache(maxsize=None)
def _mesh():
    return plsc.VectorSubcoreMesh(core_axis_name="c", subcore_axis_name="s",
                                  num_cores=NC, num_subcores=NS)


@functools.lru_cache(maxsize=None)
def _make_segsum(T, E2):
    @functools.partial(
        pl.kernel,
        out_type=jax.ShapeDtypeStruct((NC, NP, D), jnp.float32),
        mesh=_mesh(),
        scratch_types=[
            pltpu.VMEM_SHARED((NP, D), jnp.float32),
            pltpu.VMEM((CPH, C), jnp.int32),
            pltpu.VMEM((CPH, C), jnp.int32),
            pltpu.VMEM((C, D), jnp.float32),
            pltpu.VMEM((C, D), jnp.float32),
            pltpu.SemaphoreType.DMA,
            pltpu.SemaphoreType.DMA,
        ],
    )
    def segsum(table, src4, dst4, zeros, out,
               acc, sidx, didx, rows0, rows1, sem0, sem1):
        c = lax.axis_index("c")
        s = lax.axis_index("s")
        w = c * NS + s
        pltpu.sync_copy(zeros, acc.at[pl.ds(s * RPT, RPT)])
        plsc.subcore_barrier()

        rows = (rows0, rows1)
        sems = (sem0, sem1)
        for phase in range(_PH):
            pltpu.sync_copy(src4.at[w, phase], sidx)
            pltpu.sync_copy(dst4.at[w, phase], didx)
            pltpu.async_copy(table.at[sidx.at[0]], rows0, sem0)
            pltpu.async_copy(table.at[sidx.at[1]], rows1, sem1)

            def body(i, carry):
                for b in range(2):
                    j = 2 * i + b
                    pltpu.make_async_copy(table.at[sidx.at[j]],
                                          rows[b], sems[b]).wait()
                    pltpu.sync_copy(rows[b], acc.at[didx.at[j]], add=True)

                    @pl.when(j + 2 < CPH)
                    def _():
                        pltpu.async_copy(table.at[sidx.at[j + 2]],
                                         rows[b], sems[b])
                return carry

            lax.fori_loop(0, CPH // 2, body, 0)
        plsc.subcore_barrier()
        pltpu.sync_copy(acc.at[pl.ds(s * RPT, RPT)],
                        out.at[c, pl.ds(s * RPT, RPT)])

    return segsum


_CW = 128
_CCH = (E // NW) // C


@functools.lru_cache(maxsize=None)
def _make_counts():
    @functools.partial(
        pl.kernel,
        out_type=(
            jax.ShapeDtypeStruct((NC, NP, _CW), jnp.float32),
            jax.ShapeDtypeStruct((NC, NP, _CW), jnp.float32),
            jax.ShapeDtypeStruct((NC, NP, _CW), jnp.float32),
        ),
        mesh=_mesh(),
        scratch_types=[
            pltpu.VMEM_SHARED((NP, _CW), jnp.float32),
            pltpu.VMEM((CPH, C), jnp.int32),
            pltpu.VMEM((C, _CW), jnp.float32),
            pltpu.SemaphoreType.DMA,
        ],
    )
    def counts(dst4, v4, e4, zeros, ones, outd, outv, oute,
               acc, idx, ones_v, sem):
        c = lax.axis_index("c")
        s = lax.axis_index("s")
        w = c * NS + s
        sl = pl.ds(s * RPT, RPT)
        pltpu.sync_copy(ones, ones_v)
        for idx4, out in ((dst4, outd), (v4, outv), (e4, oute)):
            pltpu.sync_copy(zeros, acc.at[sl])
            plsc.subcore_barrier()
            for phase in range(_PH):
                pltpu.sync_copy(idx4.at[w, phase], idx)

                def body(j, carry):
                    @pl.when(j >= 4)
                    def _():
                        pltpu.make_async_copy(ones_v, acc.at[idx.at[j - 4]],
                                              sem).wait()

                    pltpu.async_copy(ones_v, acc.at[idx.at[j]], sem,
                                     add=True)
                    return carry

                lax.fori_loop(0, CPH, body, 0)
                for b in range(4):
                    pltpu.make_async_copy(ones_v, acc.at[idx.at[CPH - 4 + b]],
                                          sem).wait()
            plsc.subcore_barrier()
            pltpu.sync_copy(acc.at[sl], out.at[c, sl])
            plsc.subcore_barrier()

    return counts


_B = 1000
_GRID = N // _B


def _row_spec(nrow=_B, ncol=D):
    return pl.BlockSpec((nrow, ncol), lambda i: (i, 0))


def _part_spec(ncol=D):
    return pl.BlockSpec((NC, _B, ncol), lambda i: (0, i, 0))


def _full_spec(shape):
    nd = len(shape)
    return pl.BlockSpec(shape, lambda i: (0,) * nd)


def _tc1_body(x, wg1, wh1, bh1, degp, vcp, ecp,
              hg1p, hh1, dinv_b, vinv_b, einv_b):
    deg = degp[0, :, :1] + degp[1, :, :1] + 1.0
    dinv = lax.rsqrt(deg)
    dinv_b[...] = jnp.broadcast_to(dinv, (_B, D))
    vinv_b[...] = jnp.broadcast_to(
        1.0 / jnp.maximum(vcp[0, :, :1] + vcp[1, :, :1], 1.0), (_B, D))
    einv_b[...] = jnp.broadcast_to(
        1.0 / jnp.maximum(ecp[0, :, :1] + ecp[1, :, :1], 1.0), (_B, D))
    hg1p[...] = jnp.dot(x[...], wg1[...],
                        preferred_element_type=jnp.float32) * dinv_b[...]
    hh1[...] = jnp.dot(x[...], wh1[...],
                       preferred_element_type=jnp.float32) + bh1[...]


def _tc2_body(sg1p, hg1p, bg1, dinv_b, se1p, einv_b, wg2, hg2p, ef1):
    x1 = jnp.maximum(
        dinv_b[...] * (sg1p[0] + sg1p[1] + hg1p[...]) + bg1[...], 0.0)
    hg2p[...] = jnp.dot(x1, wg2[...],
                        preferred_element_type=jnp.float32) * dinv_b[...]
    ef1[...] = (se1p[0] + se1p[1]) * einv_b[...]


def _tc3_body(sg2p, hg2p, bg2, dinv_b, sv1p, vinv_b, wh2, bh2, x2, hh2):
    x2[...] = dinv_b[...] * (sg2p[0] + sg2p[1] + hg2p[...]) + bg2[...]
    x3 = jnp.maximum((sv1p[0] + sv1p[1]) * vinv_b[...], 0.0)
    hh2[...] = jnp.dot(x3, wh2[...],
                       preferred_element_type=jnp.float32) + bh2[...]


def _tc4_body(se2p, einv_b, ef2):
    ef2[...] = (se2p[0] + se2p[1]) * einv_b[...]


def _tc5_body(sv2p, vinv_b, x2, out):
    out[...] = 0.5 * x2[...] + 0.5 * (sv2p[0] + sv2p[1]) * vinv_b[...]


def _row_out(k=1):
    o = [jax.ShapeDtypeStruct((N, D), jnp.float32) for _ in range(k)]
    return o[0] if k == 1 else tuple(o)


_tc1 = pl.pallas_call(
    _tc1_body,
    grid=(_GRID,),
    in_specs=[_row_spec(), _full_spec((D, D)), _full_spec((D, D)),
              _full_spec((D,)), _part_spec(_CW), _part_spec(_CW),
              _part_spec(_CW)],
    out_specs=[_row_spec()] * 5,
    out_shape=_row_out(5),
)

_tc2 = pl.pallas_call(
    _tc2_body,
    grid=(_GRID,),
    in_specs=[_part_spec(), _row_spec(), _full_spec((D,)), _row_spec(),
              _part_spec(), _row_spec(), _full_spec((D, D))],
    out_specs=[_row_spec()] * 2,
    out_shape=_row_out(2),
)

_tc3 = pl.pallas_call(
    _tc3_body,
    grid=(_GRID,),
    in_specs=[_part_spec(), _row_spec(), _full_spec((D,)), _row_spec(),
              _part_spec(), _row_spec(), _full_spec((D, D)),
              _full_spec((D,))],
    out_specs=[_row_spec()] * 2,
    out_shape=_row_out(2),
)

_tc4 = pl.pallas_call(
    _tc4_body,
    grid=(_GRID,),
    in_specs=[_part_spec(), _row_spec()],
    out_specs=_row_spec(),
    out_shape=_row_out(),
)

_tc5 = pl.pallas_call(
    _tc5_body,
    grid=(_GRID,),
    in_specs=[_part_spec(), _row_spec(), _row_spec()],
    out_specs=_row_spec(),
    out_shape=_row_out(),
)


def kernel(x, edge_index, hyperedge_index,
           W_g1, b_g1, W_g2, b_g2, W_h1, b_h1, W_h2, b_h2):
    def chunked(a, fill):
        a = a.reshape(NW, EPT)
        pad = jnp.full((NW, EPTP - EPT), fill, a.dtype)
        return jnp.concatenate([a, pad], axis=1).reshape(NW, _PH, CPH, C)

    src2 = chunked(edge_index[0], 0)
    dst2 = chunked(edge_index[1], SINK)
    v2g = chunked(hyperedge_index[0], 0)
    v2s = chunked(hyperedge_index[0], SINK)
    e2g = chunked(hyperedge_index[1], 0)
    e2s = chunked(hyperedge_index[1], SINK)

    zeros_d = jnp.zeros((RPT, D), jnp.float32)
    zeros_c = jnp.zeros((RPT, _CW), jnp.float32)
    ones_c = jnp.ones((C, _CW), jnp.float32)

    segsum_nodes = _make_segsum(N, E)
    segsum_v2e = _make_segsum(N, NNZ)
    segsum_e2v = _make_segsum(NE, NNZ)

    degp, vcp, ecp = _make_counts()(dst2, v2s, e2s, zeros_c, ones_c)
    hg1p, hh1, dinv_b, vinv_b, einv_b = _tc1(
        x, W_g1, W_h1, b_h1, degp, vcp, ecp)

    sg1p = segsum_nodes(hg1p, src2, dst2, zeros_d)
    se1p = segsum_v2e(hh1, v2g, e2s, zeros_d)
    hg2p, ef1 = _tc2(sg1p, hg1p, b_g1, dinv_b, se1p, einv_b, W_g2)

    sg2p = segsum_nodes(hg2p, src2, dst2, zeros_d)
    sv1p = segsum_e2v(ef1, e2g, v2s, zeros_d)
    x2, hh2 = _tc3(sg2p, hg2p, b_g2, dinv_b, sv1p, vinv_b, W_h2, b_h2)

    se2p = segsum_v2e(hh2, v2g, e2s, zeros_d)
    ef2 = _tc4(se2p, einv_b)

    sv2p = segsum_e2v(ef2, e2g, v2s, zeros_d)
    return _tc5(sv2p, vinv_b, x2)


# speedup vs baseline: 3.2639x; 1.1928x over previous
import functools

import jax
import jax.numpy as jnp
from jax import lax
from jax.experimental import pallas as pl
from jax.experimental.pallas import tpu as pltpu
from jax.experimental.pallas import tpu_sc as plsc

N = 10000
E = 320000
NE = 10000
NNZ = 320000
D = 128

NC = 2
NS = 16
NW = NC * NS
C = 125
NP = 10240
RPT = NP // NS
SINK = NP - 1
EPT = E // NW
EPTP = 10000
_PH = 2
CPH = EPTP // _PH // C

@functools.lru_cache(maxsize=None)
def _mesh():
    return plsc.VectorSubcoreMesh(core_axis_name="c", subcore_axis_name="s",
                                  num_cores=NC, num_subcores=NS)


@functools.lru_cache(maxsize=None)
def _make_segsum(T, E2):
    @functools.partial(
        pl.kernel,
        out_type=jax.ShapeDtypeStruct((NC, NP, D), jnp.float32),
        mesh=_mesh(),
        scratch_types=[
            pltpu.VMEM_SHARED((NP, D), jnp.float32),
            pltpu.VMEM((CPH, C), jnp.int32),
            pltpu.VMEM((CPH, C), jnp.int32),
            pltpu.VMEM((C, D), jnp.float32),
            pltpu.VMEM((C, D), jnp.float32),
            pltpu.SemaphoreType.DMA,
            pltpu.SemaphoreType.DMA,
        ],
    )
    def segsum(table, src4, dst4, zeros, out,
               acc, sidx, didx, rows0, rows1, sem0, sem1):
        c = lax.axis_index("c")
        s = lax.axis_index("s")
        w = c * NS + s
        pltpu.sync_copy(zeros, acc.at[pl.ds(s * RPT, RPT)])
        plsc.subcore_barrier()

        rows = (rows0, rows1)
        sems = (sem0, sem1)
        for phase in range(_PH):
            pltpu.sync_copy(src4.at[w, phase], sidx)
            pltpu.sync_copy(dst4.at[w, phase], didx)
            pltpu.async_copy(table.at[sidx.at[0]], rows0, sem0)
            pltpu.async_copy(table.at[sidx.at[1]], rows1, sem1)

            def body(i, carry):
                for b in range(2):
                    j = 2 * i + b
                    pltpu.make_async_copy(table.at[sidx.at[j]],
                                          rows[b], sems[b]).wait()
                    pltpu.sync_copy(rows[b], acc.at[didx.at[j]], add=True)

                    @pl.when(j + 2 < CPH)
                    def _():
                        pltpu.async_copy(table.at[sidx.at[j + 2]],
                                         rows[b], sems[b])
                return carry

            lax.fori_loop(0, CPH // 2, body, 0)
        plsc.subcore_barrier()
        pltpu.sync_copy(acc.at[pl.ds(s * RPT, RPT)],
                        out.at[c, pl.ds(s * RPT, RPT)])

    return segsum


_CW = 128
_CR = 80


@functools.lru_cache(maxsize=None)
def _make_counts():
    @functools.partial(
        pl.kernel,
        out_type=(
            jax.ShapeDtypeStruct((NC, NP, _CW), jnp.float32),
            jax.ShapeDtypeStruct((NC, NP, _CW), jnp.float32),
            jax.ShapeDtypeStruct((NC, NP, _CW), jnp.float32),
        ),
        mesh=_mesh(),
        compiler_params=pltpu.CompilerParams(needs_layout_passes=False),
        scratch_types=[
            pltpu.VMEM_SHARED((NS, 3 * NP), jnp.float32),
            pltpu.VMEM((3 * NP,), jnp.float32),
            pltpu.VMEM((_CR, _CW), jnp.int32),
            pltpu.VMEM((NS, RPT), jnp.float32),
            pltpu.VMEM((3 * RPT,), jnp.float32),
            pltpu.VMEM((64, _CW), jnp.float32),
        ],
    )
    def counts(idxd, idxv, idxe, zeros3, outd, outv, oute,
               shared, cnt, idx, redbuf, red, outbuf):
        c = lax.axis_index("c")
        s = lax.axis_index("s")
        w = c * NS + s
        ones = jnp.ones((16,), jnp.float32)
        pltpu.sync_copy(zeros3, cnt)
        for idx4 in (idxd, idxv, idxe):
            pltpu.sync_copy(idx4.at[w], idx)

            def abody(r, carry):
                for k in range(8):
                    iv = idx[r, pl.ds(16 * k, 16)]
                    plsc.addupdate_scatter(cnt, [iv], ones)
                return carry

            lax.fori_loop(0, _CR, abody, 0)
        pltpu.sync_copy(cnt, shared.at[s])
        plsc.subcore_barrier()
        for k in range(3):
            pltpu.sync_copy(
                shared.at[:, pl.ds(k * NP + s * RPT, RPT)], redbuf)

            def rbody(p, carry):
                acc = redbuf[0, pl.ds(16 * p, 16)]
                for r in range(1, NS):
                    acc = acc + redbuf[r, pl.ds(16 * p, 16)]
                red[pl.ds(k * RPT + 16 * p, 16)] = acc
                return carry

            lax.fori_loop(0, RPT // 16, rbody, 0)
        for k, out in ((0, outd), (1, outv), (2, oute)):

            def ebody(t, carry):
                for r in range(64):
                    pos = jnp.full((16,), k * RPT + 64 * t + r, jnp.int32)
                    val = plsc.load_gather(red, [pos])
                    for kk in range(8):
                        outbuf[r, pl.ds(16 * kk, 16)] = val
                pltpu.sync_copy(outbuf,
                                out.at[c, pl.ds(s * RPT + 64 * t, 64)])
                return carry

            lax.fori_loop(0, RPT // 64, ebody, 0)

    return counts


_B = 1000
_GRID = N // _B


def _row_spec(nrow=_B, ncol=D):
    return pl.BlockSpec((nrow, ncol), lambda i: (i, 0))


def _part_spec(ncol=D):
    return pl.BlockSpec((NC, _B, ncol), lambda i: (0, i, 0))


def _full_spec(shape):
    nd = len(shape)
    return pl.BlockSpec(shape, lambda i: (0,) * nd)


def _tc1_body(x, wg1, wh1, bh1, degp, vcp, ecp,
              hg1p, hh1, dinv_b, vinv_b, einv_b):
    deg = degp[0, :, :1] + degp[1, :, :1] + 1.0
    dinv = lax.rsqrt(deg)
    dinv_b[...] = jnp.broadcast_to(dinv, (_B, D))
    vinv_b[...] = jnp.broadcast_to(
        1.0 / jnp.maximum(vcp[0, :, :1] + vcp[1, :, :1], 1.0), (_B, D))
    einv_b[...] = jnp.broadcast_to(
        1.0 / jnp.maximum(ecp[0, :, :1] + ecp[1, :, :1], 1.0), (_B, D))
    hg1p[...] = jnp.dot(x[...], wg1[...],
                        preferred_element_type=jnp.float32) * dinv_b[...]
    hh1[...] = jnp.dot(x[...], wh1[...],
                       preferred_element_type=jnp.float32) + bh1[...]


def _tc2_body(sg1p, hg1p, bg1, dinv_b, se1p, einv_b, wg2, hg2p, ef1):
    x1 = jnp.maximum(
        dinv_b[...] * (sg1p[0] + sg1p[1] + hg1p[...]) + bg1[...], 0.0)
    hg2p[...] = jnp.dot(x1, wg2[...],
                        preferred_element_type=jnp.float32) * dinv_b[...]
    ef1[...] = (se1p[0] + se1p[1]) * einv_b[...]


def _tc3_body(sg2p, hg2p, bg2, dinv_b, sv1p, vinv_b, wh2, bh2, x2, hh2):
    x2[...] = dinv_b[...] * (sg2p[0] + sg2p[1] + hg2p[...]) + bg2[...]
    x3 = jnp.maximum((sv1p[0] + sv1p[1]) * vinv_b[...], 0.0)
    hh2[...] = jnp.dot(x3, wh2[...],
                       preferred_element_type=jnp.float32) + bh2[...]


def _tc4_body(se2p, einv_b, ef2):
    ef2[...] = (se2p[0] + se2p[1]) * einv_b[...]


def _tc5_body(sv2p, vinv_b, x2, out):
    out[...] = 0.5 * x2[...] + 0.5 * (sv2p[0] + sv2p[1]) * vinv_b[...]


def _row_out(k=1):
    o = [jax.ShapeDtypeStruct((N, D), jnp.float32) for _ in range(k)]
    return o[0] if k == 1 else tuple(o)


_tc1 = pl.pallas_call(
    _tc1_body,
    grid=(_GRID,),
    in_specs=[_row_spec(), _full_spec((D, D)), _full_spec((D, D)),
              _full_spec((D,)), _part_spec(_CW), _part_spec(_CW),
              _part_spec(_CW)],
    out_specs=[_row_spec()] * 5,
    out_shape=_row_out(5),
)

_tc2 = pl.pallas_call(
    _tc2_body,
    grid=(_GRID,),
    in_specs=[_part_spec(), _row_spec(), _full_spec((D,)), _row_spec(),
              _part_spec(), _row_spec(), _full_spec((D, D))],
    out_specs=[_row_spec()] * 2,
    out_shape=_row_out(2),
)

_tc3 = pl.pallas_call(
    _tc3_body,
    grid=(_GRID,),
    in_specs=[_part_spec(), _row_spec(), _full_spec((D,)), _row_spec(),
              _part_spec(), _row_spec(), _full_spec((D, D)),
              _full_spec((D,))],
    out_specs=[_row_spec()] * 2,
    out_shape=_row_out(2),
)

_tc4 = pl.pallas_call(
    _tc4_body,
    grid=(_GRID,),
    in_specs=[_part_spec(), _row_spec()],
    out_specs=_row_spec(),
    out_shape=_row_out(),
)

_tc5 = pl.pallas_call(
    _tc5_body,
    grid=(_GRID,),
    in_specs=[_part_spec(), _row_spec(), _row_spec()],
    out_specs=_row_spec(),
    out_shape=_row_out(),
)


def kernel(x, edge_index, hyperedge_index,
           W_g1, b_g1, W_g2, b_g2, W_h1, b_h1, W_h2, b_h2):
    def chunked(a, fill):
        a = a.reshape(NW, EPT)
        pad = jnp.full((NW, EPTP - EPT), fill, a.dtype)
        return jnp.concatenate([a, pad], axis=1).reshape(NW, _PH, CPH, C)

    src2 = chunked(edge_index[0], 0)
    dst2 = chunked(edge_index[1], SINK)
    v2g = chunked(hyperedge_index[0], 0)
    v2s = chunked(hyperedge_index[0], SINK)
    e2g = chunked(hyperedge_index[1], 0)
    e2s = chunked(hyperedge_index[1], SINK)

    def chunked_c(a, off):
        a = a.reshape(NW, EPT)
        pad = jnp.full((NW, _CR * _CW - EPT), SINK, a.dtype)
        return (jnp.concatenate([a, pad], axis=1) + off).reshape(
            NW, _CR, _CW)

    idxd = chunked_c(edge_index[1], 0)
    idxv = chunked_c(hyperedge_index[0], NP)
    idxe = chunked_c(hyperedge_index[1], 2 * NP)
    zeros3 = jnp.zeros((3 * NP,), jnp.float32)

    zeros_d = jnp.zeros((RPT, D), jnp.float32)

    segsum_nodes = _make_segsum(N, E)
    segsum_v2e = _make_segsum(N, NNZ)
    segsum_e2v = _make_segsum(NE, NNZ)

    degp, vcp, ecp = _make_counts()(idxd, idxv, idxe, zeros3)
    hg1p, hh1, dinv_b, vinv_b, einv_b = _tc1(
        x, W_g1, W_h1, b_h1, degp, vcp, ecp)

    sg1p = segsum_nodes(hg1p, src2, dst2, zeros_d)
    se1p = segsum_v2e(hh1, v2g, e2s, zeros_d)
    hg2p, ef1 = _tc2(sg1p, hg1p, b_g1, dinv_b, se1p, einv_b, W_g2)

    sg2p = segsum_nodes(hg2p, src2, dst2, zeros_d)
    sv1p = segsum_e2v(ef1, e2g, v2s, zeros_d)
    x2, hh2 = _tc3(sg2p, hg2p, b_g2, dinv_b, sv1p, vinv_b, W_h2, b_h2)

    se2p = segsum_v2e(hh2, v2g, e2s, zeros_d)
    ef2 = _tc4(se2p, einv_b)

    sv2p = segsum_e2v(ef2, e2g, v2s, zeros_d)
    return _tc5(sv2p, vinv_b, x2)
